# baseline (speedup 1.0000x reference)
import functools

import jax
import jax.numpy as jnp
from jax import lax
from jax.experimental import pallas as pl
from jax.experimental.pallas import tpu as pltpu
from jax.experimental.pallas import tpu_sc as plsc

BATCH = 4096
MAX_LEN = 200
EMB = 128
EPS = 1e-8
NUM_ROWS = 101
TABLE_ROWS = 128
PAD_ID = TABLE_ROWS - 1

_R = 32


def _tc_dec_body(lens_ref, dec_ref, ts_ref):
    i = pl.program_id(0)
    lens = lens_ref[...]
    pos = lax.broadcasted_iota(jnp.int32, (_R, MAX_LEN, EMB), 1)
    mask = pos < lens[:, :, None]
    dec_ref[...] = mask.astype(jnp.float32) + EPS
    local_max = jnp.max(lens)

    @pl.when(i == 0)
    def _init():
        ts_ref[0] = local_max

    @pl.when(i > 0)
    def _acc():
        ts_ref[0] = jnp.maximum(ts_ref[0], local_max)


_tc_dec_call = pl.pallas_call(
    _tc_dec_body,
    grid=(BATCH // _R,),
    in_specs=[
        pl.BlockSpec((_R, 1), lambda i: (i, 0)),
    ],
    out_specs=[
        pl.BlockSpec((_R, MAX_LEN, EMB), lambda i: (i, 0, 0)),
        pl.BlockSpec(memory_space=pltpu.SMEM),
    ],
    out_shape=[
        jax.ShapeDtypeStruct((BATCH, MAX_LEN, EMB), jnp.float32),
        jax.ShapeDtypeStruct((1,), jnp.int32),
    ],
)

_NC, _NS = 2, 16
_NW = _NC * _NS
_B = BATCH * MAX_LEN
_BPW = _B // _NW
_CH = 80
_NCHUNK = _BPW // _CH
_NBUF = 8
_RPW = BATCH // _NW

@functools.cache
def _make_sc_gather():
    mesh = plsc.VectorSubcoreMesh(core_axis_name="c", subcore_axis_name="s")

    @functools.partial(
        pl.kernel,
        mesh=mesh,
        out_type=jax.ShapeDtypeStruct((_B, EMB), jnp.float32),
        scratch_types=[
            pltpu.VMEM((_NCHUNK, _CH), jnp.int32),
            pltpu.VMEM((_RPW,), jnp.int32),
            pltpu.VMEM((EMB,), jnp.float32),
        ]
        + [pltpu.VMEM((_CH, EMB), jnp.float32) for _ in range(_NBUF)]
        + [pltpu.VMEM_SHARED((TABLE_ROWS, EMB), jnp.float32)]
        + [pltpu.SemaphoreType.DMA for _ in range(2 * _NBUF)],
    )
    def _sc_gather(
        emb_hbm, idx_hbm, seq_hbm, out_hbm,
        idx_v, seq_v, zrow_v, *rest,
    ):
        bufs = rest[:_NBUF]
        table_sh = rest[_NBUF]
        gsems = rest[_NBUF + 1:2 * _NBUF + 1]
        ssems = rest[2 * _NBUF + 1:]
        cid = lax.axis_index("c")
        sid = lax.axis_index("s")
        wid = sid * _NC + cid
        base = wid * _BPW

        @pl.when(sid == 0)
        def _stage_table():
            pltpu.sync_copy(emb_hbm, table_sh.at[pl.ds(0, NUM_ROWS)])
            for k in range(EMB // 16):
                zrow_v[pl.ds(16 * k, 16)] = jnp.zeros((16,), jnp.float32)
            pltpu.sync_copy(zrow_v, table_sh.at[PAD_ID])

        pltpu.sync_copy(idx_hbm.at[pl.ds(wid * _NCHUNK, _NCHUNK)], idx_v)
        pltpu.sync_copy(seq_hbm.at[pl.ds(wid * _RPW, _RPW)], seq_v)

        lv = [seq_v[pl.ds(16 * k, 16)] for k in range(_RPW // 16)]

        dnums = lax.GatherDimensionNumbers(
            offset_dims=(), collapsed_slice_dims=(0,), start_index_map=(0,)
        )

        def _splat_len(b):
            k0 = b // 16
            lane = jnp.broadcast_to(b - k0 * 16, (16,))
            sel = lv[len(lv) - 1]
            for k in range(len(lv) - 2, -1, -1):
                sel = jnp.where(k0 == k, lv[k], sel)
            return lax.gather(
                sel, lane[:, None], dnums, (1,),
                mode=lax.GatherScatterMode.PROMISE_IN_BOUNDS,
            )

        def mask_one(j, b0, t0):
            l0 = _splat_len(b0)
            l1 = _splat_len(jnp.minimum(b0 + 1, _RPW - 1))
            for k in range(_CH // 16):
                t = t0 + (k * 16 + lax.iota(jnp.int32, 16))
                over = t >= MAX_LEN
                lens = jnp.where(over, l1, l0)
                tloc = jnp.where(over, t - MAX_LEN, t)
                tok = idx_v[j, pl.ds(k * 16, 16)]
                idx_v[j, pl.ds(k * 16, 16)] = jnp.where(
                    tloc < lens, tok, PAD_ID
                )
            t0n = t0 + _CH
            wrap = t0n >= MAX_LEN
            t0n = jnp.where(wrap, t0n - MAX_LEN, t0n)
            return b0 + wrap.astype(jnp.int32), t0n

        def gather(j, buf, sem):
            return pltpu.make_async_copy(table_sh.at[idx_v.at[j]], buf, sem)

        def scatter(j, buf, sem):
            return pltpu.make_async_copy(
                buf, out_hbm.at[pl.ds(base + j * _CH, _CH)], sem
            )

        b0 = jnp.int32(0)
        t0 = jnp.int32(0)
        for i in range(2 * _NBUF):
            b0, t0 = mask_one(jnp.int32(i), b0, t0)
        plsc.subcore_barrier()
        for i in range(_NBUF):
            gather(i, bufs[i], gsems[i]).start()

        def body(p, carry):
            b0, t0 = carry
            j0 = _NBUF * p
            for i in range(_NBUF):
                gather(j0 + i, bufs[i], gsems[i]).wait()
                scatter(j0 + i, bufs[i], ssems[i]).start()
            for i in range(_NBUF):
                scatter(j0 + i, bufs[i], ssems[i]).wait()
                gather(
                    lax.rem(j0 + _NBUF + i, _NCHUNK), bufs[i], gsems[i]
                ).start()
                b0, t0 = mask_one(
                    lax.rem(j0 + 2 * _NBUF + i, _NCHUNK), b0, t0
                )
            return b0, t0

        lax.fori_loop(0, _NCHUNK // _NBUF, body, (b0, t0))
        for i in range(_NBUF):
            gather(i, bufs[i], gsems[i]).wait()

    return _sc_gather


def kernel(tokens, seq_lengths, embeddings):
    lens2d = seq_lengths.reshape(BATCH, 1)
    idx2d = tokens.reshape(_NW * _NCHUNK, _CH)
    x = _make_sc_gather()(embeddings, idx2d, seq_lengths)
    x = x.reshape(BATCH, MAX_LEN, EMB)
    dec_mask, ts = _tc_dec_call(lens2d)
    return x, dec_mask, ts[0]
